# Initial kernel scaffold; baseline (speedup 1.0000x reference)
import jax
import jax.numpy as jnp
from jax.experimental import pallas as pl


def kernel(inputs, W_ih, W_hh, b_ih, b_hh, W_kp, b_kp, W_out, b_out):
    raise NotImplementedError("write your pallas kernel here")


# speedup vs baseline: 5.3803x; 5.3803x over previous
import jax
import jax.numpy as jnp
from jax.experimental import pallas as pl
from jax.experimental.pallas import tpu as pltpu

B = 512
T = 201
INPUT_DIM = 256
HIDDEN = 256
OUT_DIM = 128
KEY_DIM = 128
D_KEY = 128
D_VAL = 128
MAX_SLOTS = 512
TEMP = 0.1
NW = T - 1


def _write_kernel(x0_ref, wkpT_ref, bkp_ref, keys_ref, vals_ref, maskrow_ref):
    x0 = x0_ref[...]
    val_part = x0[:, KEY_DIM:]
    s = jnp.sum(val_part, axis=1, keepdims=True)
    do = s >= 0.5
    dof = do.astype(jnp.float32)
    rowi = jax.lax.broadcasted_iota(jnp.int32, (NW, NW), 0)
    colj = jax.lax.broadcasted_iota(jnp.int32, (NW, NW), 1)
    lower = (colj < rowi).astype(jnp.float32)
    slot = jnp.dot(lower, dof, preferred_element_type=jnp.float32)
    sloti = slot.astype(jnp.int32)
    q = (jnp.dot(x0[:, :KEY_DIM], wkpT_ref[...],
                 preferred_element_type=jnp.float32) + bkp_ref[...])
    slots_iota = jax.lax.broadcasted_iota(jnp.int32, (NW, MAX_SLOTS), 1)
    oh = ((slots_iota == sloti) & do).astype(jnp.float32)
    keys_ref[...] = jax.lax.dot_general(
        oh, q, (((0,), (0,)), ((), ())), preferred_element_type=jnp.float32)
    vals_ref[...] = jax.lax.dot_general(
        oh, val_part, (((0,), (0,)), ((), ())),
        preferred_element_type=jnp.float32)
    maskrow_ref[...] = jnp.sum(oh, axis=0, keepdims=True)


def _lstm_kernel(x_ref, wih_ref, whh_ref, b_ref, hout_ref, h_ref, c_ref):
    t = pl.program_id(0)

    @pl.when(t == 0)
    def _():
        h_ref[...] = jnp.zeros_like(h_ref)
        c_ref[...] = jnp.zeros_like(c_ref)

    x = x_ref[...]
    h = h_ref[...]
    gates = (jnp.dot(x, wih_ref[...], preferred_element_type=jnp.float32)
             + jnp.dot(h, whh_ref[...], preferred_element_type=jnp.float32)
             + b_ref[...])
    i = jax.nn.sigmoid(gates[:, :HIDDEN])
    f = jax.nn.sigmoid(gates[:, HIDDEN:2 * HIDDEN])
    g = jnp.tanh(gates[:, 2 * HIDDEN:3 * HIDDEN])
    o = jax.nn.sigmoid(gates[:, 3 * HIDDEN:])
    c = f * c_ref[...] + i * g
    hn = o * jnp.tanh(c)
    c_ref[...] = c
    h_ref[...] = hn

    @pl.when(t == T - 1)
    def _():
        hout_ref[...] = hn


def _read_out_kernel(qx_ref, keys_ref, vals_ref, maskrow_ref, wkpT_ref,
                     bkp_ref, h_ref, woh_ref, woc_ref, bout_ref, out_ref):
    q = (jnp.dot(qx_ref[...], wkpT_ref[...],
                 preferred_element_type=jnp.float32) + bkp_ref[...])
    qn = q / (jnp.sqrt(jnp.sum(q * q, axis=1, keepdims=True)) + 1e-8)
    k = keys_ref[...]
    kn = k / (jnp.sqrt(jnp.sum(k * k, axis=1, keepdims=True)) + 1e-8)
    sim = jax.lax.dot_general(
        qn, kn, (((1,), (1,)), ((), ())),
        preferred_element_type=jnp.float32)
    active = maskrow_ref[...] > 0
    logits = jnp.where(active, sim / TEMP, -1e9)
    m = jnp.max(logits, axis=1, keepdims=True)
    e = jnp.exp(logits - m)
    attn = e / jnp.sum(e, axis=1, keepdims=True)
    attn = attn * active.astype(jnp.float32)
    denom = jnp.sum(attn, axis=1, keepdims=True)
    attn = attn / jnp.where(denom > 0, denom, 1.0)
    ctx = jnp.dot(attn, vals_ref[...], preferred_element_type=jnp.float32)
    out_ref[...] = (jnp.dot(h_ref[...], woh_ref[...],
                            preferred_element_type=jnp.float32)
                    + jnp.dot(ctx, woc_ref[...],
                              preferred_element_type=jnp.float32)
                    + bout_ref[...])


def kernel(inputs, W_ih, W_hh, b_ih, b_hh, W_kp, b_kp, W_out, b_out):
    wkpT = W_kp.T
    bkp = b_kp.reshape(1, -1)

    x0 = inputs[0, :NW, :]
    keys, values, maskrow = pl.pallas_call(
        _write_kernel,
        out_shape=[
            jax.ShapeDtypeStruct((MAX_SLOTS, D_KEY), jnp.float32),
            jax.ShapeDtypeStruct((MAX_SLOTS, D_VAL), jnp.float32),
            jax.ShapeDtypeStruct((1, MAX_SLOTS), jnp.float32),
        ],
    )(x0, wkpT, bkp)

    xflat = inputs.reshape(B, T * INPUT_DIM)
    final_h = pl.pallas_call(
        _lstm_kernel,
        grid=(T,),
        in_specs=[
            pl.BlockSpec((B, INPUT_DIM), lambda t: (0, t)),
            pl.BlockSpec((INPUT_DIM, 4 * HIDDEN), lambda t: (0, 0)),
            pl.BlockSpec((HIDDEN, 4 * HIDDEN), lambda t: (0, 0)),
            pl.BlockSpec((1, 4 * HIDDEN), lambda t: (0, 0)),
        ],
        out_specs=pl.BlockSpec((B, HIDDEN), lambda t: (0, 0)),
        out_shape=jax.ShapeDtypeStruct((B, HIDDEN), jnp.float32),
        scratch_shapes=[
            pltpu.VMEM((B, HIDDEN), jnp.float32),
            pltpu.VMEM((B, HIDDEN), jnp.float32),
        ],
    )(xflat, W_ih.T, W_hh.T, (b_ih + b_hh).reshape(1, -1))

    qx = inputs[:, T - 1, :KEY_DIM]
    woT = W_out.T
    out = pl.pallas_call(
        _read_out_kernel,
        out_shape=jax.ShapeDtypeStruct((B, OUT_DIM), jnp.float32),
    )(qx, keys, values, maskrow, wkpT, bkp, final_h,
      woT[:HIDDEN], woT[HIDDEN:], b_out.reshape(1, -1))
    return out


# speedup vs baseline: 7.6415x; 1.4203x over previous
import jax
import jax.numpy as jnp
from jax.experimental import pallas as pl
from jax.experimental.pallas import tpu as pltpu

B = 512
T = 201
INPUT_DIM = 256
HIDDEN = 256
OUT_DIM = 128
KEY_DIM = 128
D_KEY = 128
D_VAL = 128
MAX_SLOTS = 512
TEMP = 0.1
NW = T - 1
TB = 8
NB = NW // TB


def _write_kernel(x0_ref, wkpT_ref, bkp_ref, keys_ref, vals_ref, maskrow_ref):
    x0 = x0_ref[...]
    val_part = x0[:, KEY_DIM:]
    s = jnp.sum(val_part, axis=1, keepdims=True)
    do = s >= 0.5
    dof = do.astype(jnp.float32)
    rowi = jax.lax.broadcasted_iota(jnp.int32, (NW, NW), 0)
    colj = jax.lax.broadcasted_iota(jnp.int32, (NW, NW), 1)
    lower = (colj < rowi).astype(jnp.float32)
    slot = jnp.dot(lower, dof, preferred_element_type=jnp.float32)
    sloti = slot.astype(jnp.int32)
    q = (jnp.dot(x0[:, :KEY_DIM], wkpT_ref[...],
                 preferred_element_type=jnp.float32) + bkp_ref[...])
    slots_iota = jax.lax.broadcasted_iota(jnp.int32, (NW, MAX_SLOTS), 1)
    oh = ((slots_iota == sloti) & do).astype(jnp.float32)
    keys_ref[...] = jax.lax.dot_general(
        oh, q, (((0,), (0,)), ((), ())), preferred_element_type=jnp.float32)
    vals_ref[...] = jax.lax.dot_general(
        oh, val_part, (((0,), (0,)), ((), ())),
        preferred_element_type=jnp.float32)
    maskrow_ref[...] = jnp.sum(oh, axis=0, keepdims=True)


def _lstm_step(x, h, c, wih, whh, b):
    gates = (jnp.dot(x, wih, preferred_element_type=jnp.float32)
             + jnp.dot(h, whh, preferred_element_type=jnp.float32) + b)
    i = jax.nn.sigmoid(gates[:, :HIDDEN])
    f = jax.nn.sigmoid(gates[:, HIDDEN:2 * HIDDEN])
    g = jnp.tanh(gates[:, 2 * HIDDEN:3 * HIDDEN])
    o = jax.nn.sigmoid(gates[:, 3 * HIDDEN:])
    c = f * c + i * g
    h = o * jnp.tanh(c)
    return h, c


def _lstm_kernel(x_ref, wih_ref, whh_ref, b_ref, hout_ref, cout_ref,
                 h_ref, c_ref):
    j = pl.program_id(0)

    @pl.when(j == 0)
    def _():
        h_ref[...] = jnp.zeros_like(h_ref)
        c_ref[...] = jnp.zeros_like(c_ref)

    h = h_ref[...]
    c = c_ref[...]
    wih = wih_ref[...]
    whh = whh_ref[...]
    b = b_ref[...]
    for k in range(TB):
        h, c = _lstm_step(x_ref[:, k, :], h, c, wih, whh, b)
    h_ref[...] = h
    c_ref[...] = c

    @pl.when(j == NB - 1)
    def _():
        hout_ref[...] = h
        cout_ref[...] = c


def _read_out_kernel(xlast_ref, h_ref, c_ref, wih_ref, whh_ref, b_ref,
                     keys_ref, vals_ref, maskrow_ref, wkpT_ref, bkp_ref,
                     woh_ref, woc_ref, bout_ref, out_ref):
    xlast = xlast_ref[...]
    final_h, _ = _lstm_step(xlast, h_ref[...], c_ref[...],
                            wih_ref[...], whh_ref[...], b_ref[...])
    q = (jnp.dot(xlast[:, :KEY_DIM], wkpT_ref[...],
                 preferred_element_type=jnp.float32) + bkp_ref[...])
    qn = q / (jnp.sqrt(jnp.sum(q * q, axis=1, keepdims=True)) + 1e-8)
    k = keys_ref[...]
    kn = k / (jnp.sqrt(jnp.sum(k * k, axis=1, keepdims=True)) + 1e-8)
    sim = jax.lax.dot_general(
        qn, kn, (((1,), (1,)), ((), ())),
        preferred_element_type=jnp.float32)
    active = maskrow_ref[...] > 0
    logits = jnp.where(active, sim / TEMP, -1e9)
    m = jnp.max(logits, axis=1, keepdims=True)
    e = jnp.exp(logits - m)
    attn = e / jnp.sum(e, axis=1, keepdims=True)
    attn = attn * active.astype(jnp.float32)
    denom = jnp.sum(attn, axis=1, keepdims=True)
    attn = attn / jnp.where(denom > 0, denom, 1.0)
    ctx = jnp.dot(attn, vals_ref[...], preferred_element_type=jnp.float32)
    out_ref[...] = (jnp.dot(final_h, woh_ref[...],
                            preferred_element_type=jnp.float32)
                    + jnp.dot(ctx, woc_ref[...],
                              preferred_element_type=jnp.float32)
                    + bout_ref[...])


def kernel(inputs, W_ih, W_hh, b_ih, b_hh, W_kp, b_kp, W_out, b_out):
    wkpT = W_kp.T
    bkp = b_kp.reshape(1, -1)

    x0 = inputs[0, :NW, :]
    keys, values, maskrow = pl.pallas_call(
        _write_kernel,
        out_shape=[
            jax.ShapeDtypeStruct((MAX_SLOTS, D_KEY), jnp.float32),
            jax.ShapeDtypeStruct((MAX_SLOTS, D_VAL), jnp.float32),
            jax.ShapeDtypeStruct((1, MAX_SLOTS), jnp.float32),
        ],
    )(x0, wkpT, bkp)

    wihT = W_ih.T
    whhT = W_hh.T
    b2 = (b_ih + b_hh).reshape(1, -1)
    h200, c200 = pl.pallas_call(
        _lstm_kernel,
        grid=(NB,),
        in_specs=[
            pl.BlockSpec((B, TB, INPUT_DIM), lambda j: (0, j, 0)),
            pl.BlockSpec((INPUT_DIM, 4 * HIDDEN), lambda j: (0, 0)),
            pl.BlockSpec((HIDDEN, 4 * HIDDEN), lambda j: (0, 0)),
            pl.BlockSpec((1, 4 * HIDDEN), lambda j: (0, 0)),
        ],
        out_specs=[
            pl.BlockSpec((B, HIDDEN), lambda j: (0, 0)),
            pl.BlockSpec((B, HIDDEN), lambda j: (0, 0)),
        ],
        out_shape=[
            jax.ShapeDtypeStruct((B, HIDDEN), jnp.float32),
            jax.ShapeDtypeStruct((B, HIDDEN), jnp.float32),
        ],
        scratch_shapes=[
            pltpu.VMEM((B, HIDDEN), jnp.float32),
            pltpu.VMEM((B, HIDDEN), jnp.float32),
        ],
    )(inputs, wihT, whhT, b2)

    xlast = inputs[:, T - 1, :]
    woT = W_out.T
    out = pl.pallas_call(
        _read_out_kernel,
        out_shape=jax.ShapeDtypeStruct((B, OUT_DIM), jnp.float32),
    )(xlast, h200, c200, wihT, whhT, b2, keys, values, maskrow,
      wkpT, bkp, woT[:HIDDEN], woT[HIDDEN:], b_out.reshape(1, -1))
    return out


# speedup vs baseline: 8.6382x; 1.1304x over previous
import jax
import jax.numpy as jnp
from jax.experimental import pallas as pl
from jax.experimental.pallas import tpu as pltpu

B = 512
T = 201
INPUT_DIM = 256
HIDDEN = 256
OUT_DIM = 128
KEY_DIM = 128
D_KEY = 128
D_VAL = 128
MAX_SLOTS = 512
TEMP = 0.1
NW = T - 1
TB = 8
NB = NW // TB


def _write_kernel(x0_ref, wkpT_ref, bkp_ref, keys_ref, vals_ref, maskrow_ref):
    x0 = x0_ref[...]
    val_part = x0[:, KEY_DIM:]
    s = jnp.sum(val_part, axis=1, keepdims=True)
    do = s >= 0.5
    dof = do.astype(jnp.float32)
    rowi = jax.lax.broadcasted_iota(jnp.int32, (NW, NW), 0)
    colj = jax.lax.broadcasted_iota(jnp.int32, (NW, NW), 1)
    lower = (colj < rowi).astype(jnp.float32)
    slot = jnp.dot(lower, dof, preferred_element_type=jnp.float32)
    sloti = slot.astype(jnp.int32)
    q = (jnp.dot(x0[:, :KEY_DIM], wkpT_ref[...],
                 preferred_element_type=jnp.float32) + bkp_ref[...])
    slots_iota = jax.lax.broadcasted_iota(jnp.int32, (NW, MAX_SLOTS), 1)
    oh = ((slots_iota == sloti) & do).astype(jnp.float32)
    keys_ref[...] = jax.lax.dot_general(
        oh, q, (((0,), (0,)), ((), ())), preferred_element_type=jnp.float32)
    vals_ref[...] = jax.lax.dot_general(
        oh, val_part, (((0,), (0,)), ((), ())),
        preferred_element_type=jnp.float32)
    maskrow_ref[...] = jnp.sum(oh, axis=0, keepdims=True)


def _lstm_step(x, h, c, wih, whh, b):
    gates = (jnp.dot(x, wih, preferred_element_type=jnp.float32)
             + jnp.dot(h, whh, preferred_element_type=jnp.float32) + b)
    i = jnp.tanh(gates[:, :HIDDEN]) * 0.5 + 0.5
    f = jnp.tanh(gates[:, HIDDEN:2 * HIDDEN]) * 0.5 + 0.5
    g = jnp.tanh(gates[:, 2 * HIDDEN:3 * HIDDEN])
    o = jnp.tanh(gates[:, 3 * HIDDEN:]) * 0.5 + 0.5
    c = f * c + i * g
    h = o * jnp.tanh(c)
    return h, c


def _lstm_kernel(x_ref, wih_ref, whh_ref, b_ref, hout_ref, cout_ref,
                 h_ref, c_ref):
    j = pl.program_id(0)

    @pl.when(j == 0)
    def _():
        h_ref[...] = jnp.zeros_like(h_ref)
        c_ref[...] = jnp.zeros_like(c_ref)

    h = h_ref[...]
    c = c_ref[...]
    wih = wih_ref[...]
    whh = whh_ref[...]
    b = b_ref[...]
    for k in range(TB):
        h, c = _lstm_step(x_ref[:, k, :], h, c, wih, whh, b)
    h_ref[...] = h
    c_ref[...] = c

    @pl.when(j == NB - 1)
    def _():
        hout_ref[...] = h
        cout_ref[...] = c


def _read_out_kernel(xlast_ref, h_ref, c_ref, wih_ref, whh_ref, b_ref,
                     keys_ref, vals_ref, maskrow_ref, wkpT_ref, bkp_ref,
                     woh_ref, woc_ref, bout_ref, out_ref):
    xlast = xlast_ref[...]
    final_h, _ = _lstm_step(xlast, h_ref[...], c_ref[...],
                            wih_ref[...], whh_ref[...], b_ref[...])
    q = (jnp.dot(xlast[:, :KEY_DIM], wkpT_ref[...],
                 preferred_element_type=jnp.float32) + bkp_ref[...])
    qn = q / (jnp.sqrt(jnp.sum(q * q, axis=1, keepdims=True)) + 1e-8)
    k = keys_ref[...]
    kn = k / (jnp.sqrt(jnp.sum(k * k, axis=1, keepdims=True)) + 1e-8)
    sim = jax.lax.dot_general(
        qn, kn, (((1,), (1,)), ((), ())),
        preferred_element_type=jnp.float32)
    active = maskrow_ref[...] > 0
    logits = jnp.where(active, sim / TEMP, -1e9)
    m = jnp.max(logits, axis=1, keepdims=True)
    e = jnp.exp(logits - m)
    attn = e / jnp.sum(e, axis=1, keepdims=True)
    attn = attn * active.astype(jnp.float32)
    denom = jnp.sum(attn, axis=1, keepdims=True)
    attn = attn / jnp.where(denom > 0, denom, 1.0)
    ctx = jnp.dot(attn, vals_ref[...], preferred_element_type=jnp.float32)
    out_ref[...] = (jnp.dot(final_h, woh_ref[...],
                            preferred_element_type=jnp.float32)
                    + jnp.dot(ctx, woc_ref[...],
                              preferred_element_type=jnp.float32)
                    + bout_ref[...])


def kernel(inputs, W_ih, W_hh, b_ih, b_hh, W_kp, b_kp, W_out, b_out):
    wkpT = W_kp.T
    bkp = b_kp.reshape(1, -1)

    x0 = inputs[0, :NW, :]
    keys, values, maskrow = pl.pallas_call(
        _write_kernel,
        out_shape=[
            jax.ShapeDtypeStruct((MAX_SLOTS, D_KEY), jnp.float32),
            jax.ShapeDtypeStruct((MAX_SLOTS, D_VAL), jnp.float32),
            jax.ShapeDtypeStruct((1, MAX_SLOTS), jnp.float32),
        ],
    )(x0, wkpT, bkp)

    gsc = jnp.concatenate([
        jnp.full((2 * HIDDEN,), 0.5, jnp.float32),
        jnp.ones((HIDDEN,), jnp.float32),
        jnp.full((HIDDEN,), 0.5, jnp.float32)])
    wihT = W_ih.T * gsc
    whhT = W_hh.T * gsc
    b2 = ((b_ih + b_hh) * gsc).reshape(1, -1)
    h200, c200 = pl.pallas_call(
        _lstm_kernel,
        grid=(NB,),
        in_specs=[
            pl.BlockSpec((B, TB, INPUT_DIM), lambda j: (0, j, 0)),
            pl.BlockSpec((INPUT_DIM, 4 * HIDDEN), lambda j: (0, 0)),
            pl.BlockSpec((HIDDEN, 4 * HIDDEN), lambda j: (0, 0)),
            pl.BlockSpec((1, 4 * HIDDEN), lambda j: (0, 0)),
        ],
        out_specs=[
            pl.BlockSpec((B, HIDDEN), lambda j: (0, 0)),
            pl.BlockSpec((B, HIDDEN), lambda j: (0, 0)),
        ],
        out_shape=[
            jax.ShapeDtypeStruct((B, HIDDEN), jnp.float32),
            jax.ShapeDtypeStruct((B, HIDDEN), jnp.float32),
        ],
        scratch_shapes=[
            pltpu.VMEM((B, HIDDEN), jnp.float32),
            pltpu.VMEM((B, HIDDEN), jnp.float32),
        ],
    )(inputs, wihT, whhT, b2)

    xlast = inputs[:, T - 1, :]
    woT = W_out.T
    out = pl.pallas_call(
        _read_out_kernel,
        out_shape=jax.ShapeDtypeStruct((B, OUT_DIM), jnp.float32),
    )(xlast, h200, c200, wihT, whhT, b2, keys, values, maskrow,
      wkpT, bkp, woT[:HIDDEN], woT[HIDDEN:], b_out.reshape(1, -1))
    return out


# speedup vs baseline: 8.6646x; 1.0030x over previous
import jax
import jax.numpy as jnp
from jax.experimental import pallas as pl
from jax.experimental.pallas import tpu as pltpu

B = 512
T = 201
INPUT_DIM = 256
HIDDEN = 256
OUT_DIM = 128
KEY_DIM = 128
D_KEY = 128
D_VAL = 128
MAX_SLOTS = 512
TEMP = 0.1
NW = T - 1
TB = 40
NB = NW // TB


def _write_kernel(x0_ref, wkpT_ref, bkp_ref, keys_ref, vals_ref, maskrow_ref):
    x0 = x0_ref[...]
    val_part = x0[:, KEY_DIM:]
    s = jnp.sum(val_part, axis=1, keepdims=True)
    do = s >= 0.5
    dof = do.astype(jnp.float32)
    rowi = jax.lax.broadcasted_iota(jnp.int32, (NW, NW), 0)
    colj = jax.lax.broadcasted_iota(jnp.int32, (NW, NW), 1)
    lower = (colj < rowi).astype(jnp.float32)
    slot = jnp.dot(lower, dof, preferred_element_type=jnp.float32)
    sloti = slot.astype(jnp.int32)
    q = (jnp.dot(x0[:, :KEY_DIM], wkpT_ref[...],
                 preferred_element_type=jnp.float32) + bkp_ref[...])
    slots_iota = jax.lax.broadcasted_iota(jnp.int32, (NW, MAX_SLOTS), 1)
    oh = ((slots_iota == sloti) & do).astype(jnp.float32)
    keys_ref[...] = jax.lax.dot_general(
        oh, q, (((0,), (0,)), ((), ())), preferred_element_type=jnp.float32)
    vals_ref[...] = jax.lax.dot_general(
        oh, val_part, (((0,), (0,)), ((), ())),
        preferred_element_type=jnp.float32)
    maskrow_ref[...] = jnp.sum(oh, axis=0, keepdims=True)


def _lstm_step(x, h, c, wih, whh, b):
    gates = (jnp.dot(x, wih, preferred_element_type=jnp.float32)
             + jnp.dot(h, whh, preferred_element_type=jnp.float32) + b)
    i = jnp.tanh(gates[:, :HIDDEN]) * 0.5 + 0.5
    f = jnp.tanh(gates[:, HIDDEN:2 * HIDDEN]) * 0.5 + 0.5
    g = jnp.tanh(gates[:, 2 * HIDDEN:3 * HIDDEN])
    o = jnp.tanh(gates[:, 3 * HIDDEN:]) * 0.5 + 0.5
    c = f * c + i * g
    h = o * jnp.tanh(c)
    return h, c


def _lstm_kernel(x_ref, wih_ref, whh_ref, b_ref, hout_ref, cout_ref,
                 h_ref, c_ref):
    j = pl.program_id(0)

    @pl.when(j == 0)
    def _():
        h_ref[...] = jnp.zeros_like(h_ref)
        c_ref[...] = jnp.zeros_like(c_ref)

    h = h_ref[...]
    c = c_ref[...]
    wih = wih_ref[...]
    whh = whh_ref[...]
    b = b_ref[...]
    for k in range(TB):
        h, c = _lstm_step(x_ref[:, k, :], h, c, wih, whh, b)
    h_ref[...] = h
    c_ref[...] = c

    @pl.when(j == NB - 1)
    def _():
        hout_ref[...] = h
        cout_ref[...] = c


def _read_out_kernel(xlast_ref, h_ref, c_ref, wih_ref, whh_ref, b_ref,
                     keys_ref, vals_ref, maskrow_ref, wkpT_ref, bkp_ref,
                     woh_ref, woc_ref, bout_ref, out_ref):
    xlast = xlast_ref[...]
    final_h, _ = _lstm_step(xlast, h_ref[...], c_ref[...],
                            wih_ref[...], whh_ref[...], b_ref[...])
    q = (jnp.dot(xlast[:, :KEY_DIM], wkpT_ref[...],
                 preferred_element_type=jnp.float32) + bkp_ref[...])
    qn = q / (jnp.sqrt(jnp.sum(q * q, axis=1, keepdims=True)) + 1e-8)
    k = keys_ref[...]
    kn = k / (jnp.sqrt(jnp.sum(k * k, axis=1, keepdims=True)) + 1e-8)
    sim = jax.lax.dot_general(
        qn, kn, (((1,), (1,)), ((), ())),
        preferred_element_type=jnp.float32)
    active = maskrow_ref[...] > 0
    logits = jnp.where(active, sim / TEMP, -1e9)
    m = jnp.max(logits, axis=1, keepdims=True)
    e = jnp.exp(logits - m)
    attn = e / jnp.sum(e, axis=1, keepdims=True)
    attn = attn * active.astype(jnp.float32)
    denom = jnp.sum(attn, axis=1, keepdims=True)
    attn = attn / jnp.where(denom > 0, denom, 1.0)
    ctx = jnp.dot(attn, vals_ref[...], preferred_element_type=jnp.float32)
    out_ref[...] = (jnp.dot(final_h, woh_ref[...],
                            preferred_element_type=jnp.float32)
                    + jnp.dot(ctx, woc_ref[...],
                              preferred_element_type=jnp.float32)
                    + bout_ref[...])


def kernel(inputs, W_ih, W_hh, b_ih, b_hh, W_kp, b_kp, W_out, b_out):
    wkpT = W_kp.T
    bkp = b_kp.reshape(1, -1)

    x0 = inputs[0, :NW, :]
    keys, values, maskrow = pl.pallas_call(
        _write_kernel,
        out_shape=[
            jax.ShapeDtypeStruct((MAX_SLOTS, D_KEY), jnp.float32),
            jax.ShapeDtypeStruct((MAX_SLOTS, D_VAL), jnp.float32),
            jax.ShapeDtypeStruct((1, MAX_SLOTS), jnp.float32),
        ],
    )(x0, wkpT, bkp)

    gsc = jnp.concatenate([
        jnp.full((2 * HIDDEN,), 0.5, jnp.float32),
        jnp.ones((HIDDEN,), jnp.float32),
        jnp.full((HIDDEN,), 0.5, jnp.float32)])
    wihT = W_ih.T * gsc
    whhT = W_hh.T * gsc
    b2 = ((b_ih + b_hh) * gsc).reshape(1, -1)
    h200, c200 = pl.pallas_call(
        _lstm_kernel,
        grid=(NB,),
        in_specs=[
            pl.BlockSpec((B, TB, INPUT_DIM), lambda j: (0, j, 0)),
            pl.BlockSpec((INPUT_DIM, 4 * HIDDEN), lambda j: (0, 0)),
            pl.BlockSpec((HIDDEN, 4 * HIDDEN), lambda j: (0, 0)),
            pl.BlockSpec((1, 4 * HIDDEN), lambda j: (0, 0)),
        ],
        out_specs=[
            pl.BlockSpec((B, HIDDEN), lambda j: (0, 0)),
            pl.BlockSpec((B, HIDDEN), lambda j: (0, 0)),
        ],
        out_shape=[
            jax.ShapeDtypeStruct((B, HIDDEN), jnp.float32),
            jax.ShapeDtypeStruct((B, HIDDEN), jnp.float32),
        ],
        scratch_shapes=[
            pltpu.VMEM((B, HIDDEN), jnp.float32),
            pltpu.VMEM((B, HIDDEN), jnp.float32),
        ],
    )(inputs, wihT, whhT, b2)

    xlast = inputs[:, T - 1, :]
    woT = W_out.T
    out = pl.pallas_call(
        _read_out_kernel,
        out_shape=jax.ShapeDtypeStruct((B, OUT_DIM), jnp.float32),
    )(xlast, h200, c200, wihT, whhT, b2, keys, values, maskrow,
      wkpT, bkp, woT[:HIDDEN], woT[HIDDEN:], b_out.reshape(1, -1))
    return out
